# baseline (speedup 1.0000x reference)
import functools

import jax
import jax.numpy as jnp
from jax import lax
from jax.experimental import pallas as pl
from jax.experimental.pallas import tpu as pltpu
from jax.experimental.pallas import tpu_sc as plsc

BATCH = 4096
SEQ = 200
EMB_DIM = 64

NC = 2
NS = 16
NW = NC * NS

B_TOTAL = BATCH * SEQ
B_PER_W = B_TOTAL // NW
IDX_MINOR = 128
N_STREAMS = B_PER_W // IDX_MINOR
GROUP = 4
N_ITERS = N_STREAMS // GROUP
ROWS_PER_ITER = GROUP * IDX_MINOR


def _make_kernel():
    mesh = plsc.VectorSubcoreMesh(core_axis_name="c", subcore_axis_name="s")

    @functools.partial(
        pl.kernel,
        out_type=jax.ShapeDtypeStruct((B_TOTAL, EMB_DIM), jnp.float32),
        mesh=mesh,
        scratch_types=[
            pltpu.VMEM((N_STREAMS, IDX_MINOR), jnp.int32),
            pltpu.VMEM((ROWS_PER_ITER, EMB_DIM), jnp.float32),
            pltpu.SemaphoreType.DMA,
        ],
        compiler_params=pltpu.CompilerParams(use_tc_tiling_on_sc=False),
    )
    def emb(x_hbm, table_hbm, out_hbm, idx_v, rows_v, sem):
        wid = lax.axis_index("s") * NC + lax.axis_index("c")
        base = wid * B_PER_W
        pltpu.sync_copy(x_hbm.at[wid], idx_v)

        def body(i, carry):
            copies = []
            for j in range(GROUP):
                c = pltpu.async_copy(
                    table_hbm.at[idx_v.at[i * GROUP + j]],
                    rows_v.at[pl.ds(j * IDX_MINOR, IDX_MINOR), :],
                    sem,
                )
                copies.append(c)
            for c in copies:
                c.wait()
            pltpu.sync_copy(
                rows_v,
                out_hbm.at[pl.ds(base + i * ROWS_PER_ITER, ROWS_PER_ITER), :],
            )
            return carry

        lax.fori_loop(0, N_ITERS, body, 0)

    return emb


_emb = _make_kernel()


@jax.jit
def kernel(x, table):
    x_r = x.reshape(NW, N_STREAMS, IDX_MINOR).astype(jnp.int32)
    out = _emb(x_r, table)
    return out.reshape(BATCH, SEQ, EMB_DIM)


# speedup vs baseline: 1.0263x; 1.0263x over previous
import functools

import jax
import jax.numpy as jnp
from jax import lax
from jax.experimental import pallas as pl
from jax.experimental.pallas import tpu as pltpu
from jax.experimental.pallas import tpu_sc as plsc

BATCH = 4096
SEQ = 200
EMB_DIM = 64

NC = 2
NS = 16
NW = NC * NS

B_TOTAL = BATCH * SEQ
B_PER_W = B_TOTAL // NW
IDX_MINOR = 128
N_STREAMS = B_PER_W // IDX_MINOR
GROUP = 4
ROWS_PER_CHUNK = GROUP * IDX_MINOR
N_CHUNKS = N_STREAMS // GROUP
N_HALF = N_CHUNKS // 2


def _make_kernel():
    mesh = plsc.VectorSubcoreMesh(core_axis_name="c", subcore_axis_name="s")

    @functools.partial(
        pl.kernel,
        out_type=jax.ShapeDtypeStruct((B_TOTAL, EMB_DIM), jnp.float32),
        mesh=mesh,
        scratch_types=[
            pltpu.VMEM((N_STREAMS, IDX_MINOR), jnp.int32),
            pltpu.VMEM((ROWS_PER_CHUNK, EMB_DIM), jnp.float32),
            pltpu.VMEM((ROWS_PER_CHUNK, EMB_DIM), jnp.float32),
            pltpu.SemaphoreType.DMA,
            pltpu.SemaphoreType.DMA,
        ],
        compiler_params=pltpu.CompilerParams(use_tc_tiling_on_sc=False),
    )
    def emb(x_hbm, table_hbm, out_hbm, idx_v, buf_a, buf_b, sem_a, sem_b):
        wid = lax.axis_index("s") * NC + lax.axis_index("c")
        base = wid * B_PER_W
        pltpu.sync_copy(x_hbm.at[wid], idx_v)

        def fire(c, buf, sem):
            for j in range(GROUP):
                pltpu.async_copy(
                    table_hbm.at[idx_v.at[c * GROUP + j]],
                    buf.at[pl.ds(j * IDX_MINOR, IDX_MINOR), :],
                    sem,
                )

        def drain(c, buf, sem):
            for j in range(GROUP):
                pltpu.make_async_copy(
                    table_hbm.at[idx_v.at[c * GROUP + j]],
                    buf.at[pl.ds(j * IDX_MINOR, IDX_MINOR), :],
                    sem,
                ).wait()

        def writeback(c, buf):
            pltpu.sync_copy(
                buf,
                out_hbm.at[pl.ds(base + c * ROWS_PER_CHUNK, ROWS_PER_CHUNK), :],
            )

        fire(0, buf_a, sem_a)

        def body(k, carry):
            c0 = 2 * k
            fire(c0 + 1, buf_b, sem_b)
            drain(c0, buf_a, sem_a)
            writeback(c0, buf_a)

            @pl.when(k < N_HALF - 1)
            def _():
                fire(c0 + 2, buf_a, sem_a)

            drain(c0 + 1, buf_b, sem_b)
            writeback(c0 + 1, buf_b)
            return carry

        lax.fori_loop(0, N_HALF, body, 0)

    return emb


_emb = _make_kernel()


@jax.jit
def kernel(x, table):
    x_r = x.reshape(NW, N_STREAMS, IDX_MINOR).astype(jnp.int32)
    out = _emb(x_r, table)
    return out.reshape(BATCH, SEQ, EMB_DIM)


# speedup vs baseline: 1.2536x; 1.2215x over previous
import functools

import jax
import jax.numpy as jnp
from jax import lax
from jax.experimental import pallas as pl
from jax.experimental.pallas import tpu as pltpu
from jax.experimental.pallas import tpu_sc as plsc

BATCH = 4096
SEQ = 200
EMB_DIM = 64
PAD_DIM = 128
VOCAB_ROWS = 1000000

NC = 2
NS = 16
NW = NC * NS

B_TOTAL = BATCH * SEQ
B_PER_W = B_TOTAL // NW
IDX_MINOR = 128
N_STREAMS = B_PER_W // IDX_MINOR
GROUP = 2
ROWS_PER_CHUNK = GROUP * IDX_MINOR
N_CHUNKS = N_STREAMS // GROUP
N_HALF = N_CHUNKS // 2


def _make_kernel():
    mesh = plsc.VectorSubcoreMesh(core_axis_name="c", subcore_axis_name="s")

    @functools.partial(
        pl.kernel,
        out_type=jax.ShapeDtypeStruct((B_TOTAL, PAD_DIM), jnp.float32),
        mesh=mesh,
        scratch_types=[
            pltpu.VMEM((N_STREAMS, IDX_MINOR), jnp.int32),
            pltpu.VMEM((ROWS_PER_CHUNK, PAD_DIM), jnp.float32),
            pltpu.VMEM((ROWS_PER_CHUNK, PAD_DIM), jnp.float32),
            pltpu.SemaphoreType.DMA,
            pltpu.SemaphoreType.DMA,
        ],
        compiler_params=pltpu.CompilerParams(use_tc_tiling_on_sc=False),
    )
    def emb(x_hbm, table_hbm, out_hbm, idx_v, buf_a, buf_b, sem_a, sem_b):
        wid = lax.axis_index("s") * NC + lax.axis_index("c")
        base = wid * B_PER_W
        pltpu.sync_copy(x_hbm.at[wid], idx_v)

        def fire(c, buf, sem):
            for j in range(GROUP):
                pltpu.async_copy(
                    table_hbm.at[idx_v.at[c * GROUP + j]],
                    buf.at[pl.ds(j * IDX_MINOR, IDX_MINOR), :],
                    sem,
                )

        def drain(c, buf, sem):
            for j in range(GROUP):
                pltpu.make_async_copy(
                    table_hbm.at[idx_v.at[c * GROUP + j]],
                    buf.at[pl.ds(j * IDX_MINOR, IDX_MINOR), :],
                    sem,
                ).wait()

        def writeback(c, buf):
            pltpu.sync_copy(
                buf,
                out_hbm.at[pl.ds(base + c * ROWS_PER_CHUNK, ROWS_PER_CHUNK), :],
            )

        fire(0, buf_a, sem_a)

        def body(k, carry):
            c0 = 2 * k
            fire(c0 + 1, buf_b, sem_b)
            drain(c0, buf_a, sem_a)
            writeback(c0, buf_a)

            @pl.when(k < N_HALF - 1)
            def _():
                fire(c0 + 2, buf_a, sem_a)

            drain(c0 + 1, buf_b, sem_b)
            writeback(c0 + 1, buf_b)
            return carry

        lax.fori_loop(0, N_HALF, body, 0)

    return emb


_emb = _make_kernel()


@jax.jit
def kernel(x, table):
    x_r = x.reshape(NW, N_STREAMS, IDX_MINOR).astype(jnp.int32)
    table_p = jnp.pad(table, ((0, 0), (0, PAD_DIM - EMB_DIM)))
    out_p = _emb(x_r, table_p)
    return out_p[:, :EMB_DIM].reshape(BATCH, SEQ, EMB_DIM)
